# baseline (speedup 1.0000x reference)
import functools

import jax
import jax.numpy as jnp
from jax import lax
from jax.experimental import pallas as pl
from jax.experimental.pallas import tpu as pltpu
from jax.experimental.pallas import tpu_sc as plsc

_G = 64
_NCAND = 1024
_NEDGE = 2048
_K = 32
_NDEL = _G * _NEDGE
_NADD = _G * _NCAND
_NW = 32
_L = 16


_GDN = lax.GatherDimensionNumbers(
    offset_dims=(), collapsed_slice_dims=(0,), start_index_map=(0,))


def _vgather(v, idx):
    return lax.gather(v, idx, _GDN, (1,),
                      mode=lax.GatherScatterMode.PROMISE_IN_BOUNDS)


def _butterfly(v, op):
    lane = lax.iota(jnp.int32, _L)
    for sh in (8, 4, 2, 1):
        v = op(v, _vgather(v, (lane ^ sh)[:, None]))
    return v


def _keys16(x):
    b = lax.bitcast_convert_type(x, jnp.int32)
    return jnp.where(b >= 0, b, b ^ 0x7FFFFFFF)


def _kth_largest(key_ref, nvec, k):
    U = 16

    def count_ge(cand):
        def step(j, acc):
            for u in range(U):
                v = key_ref[pl.ds(pl.multiple_of((j * U + u) * _L, _L), _L)]
                acc = acc + jnp.where(v >= cand, 1, 0)
            return acc
        acc = lax.fori_loop(0, nvec // U, step, jnp.zeros((_L,), jnp.int32))
        return _butterfly(acc, jnp.add)

    zero = jnp.zeros((_L,), jnp.int32)
    T = jnp.where(count_ge(zero) >= k, 0, -2**31).astype(jnp.int32)

    def bit_step(i, T):
        cand = T | (1 << (30 - i))
        return jnp.where(count_ge(cand) >= k, cand, T)
    return lax.fori_loop(0, 31, bit_step, T)


def _sc_body(addl, dell, out_w, dbuf, dkey, abuf, akey):
    wid = lax.axis_index("s")

    def do_graph(t, _):
        g = wid * 4 + t

        pltpu.sync_copy(dell.at[pl.ds(g * _NEDGE, _NEDGE)], dbuf)
        nd = _NEDGE // _L

        def dk_step(j, _):
            for u in range(4):
                sl = pl.ds(pl.multiple_of((j * 4 + u) * _L, _L), _L)
                dkey[sl] = ~_keys16(dbuf[sl])
            return 0
        lax.fori_loop(0, nd // 4, dk_step, 0)
        Td = _kth_largest(dkey, nd, _K)

        def dw_step(j, _):
            for u in range(4):
                sl = pl.ds(pl.multiple_of((j * 4 + u) * _L, _L), _L)
                dbuf[sl] = jnp.where(dkey[sl] >= Td, 0.0, 1.0)
            return 0
        lax.fori_loop(0, nd // 4, dw_step, 0)
        pltpu.sync_copy(dbuf, out_w.at[pl.ds(g * _NEDGE, _NEDGE)])

        pltpu.sync_copy(addl.at[pl.ds(g * _NCAND, _NCAND)], abuf)
        na = _NCAND // _L

        def ak_step(j, acc):
            for u in range(4):
                sl = pl.ds(pl.multiple_of((j * 4 + u) * _L, _L), _L)
                x = abuf[sl]
                akey[sl] = _keys16(x)
                acc = jnp.maximum(acc, x)
            return acc
        m16 = lax.fori_loop(0, na // 4, ak_step,
                            jnp.full((_L,), -jnp.inf, jnp.float32))
        m = _butterfly(m16, jnp.maximum)
        Ta = _kth_largest(akey, na, _K)

        def exp_step(j, acc):
            for u in range(4):
                sl = pl.ds(pl.multiple_of((j * 4 + u) * _L, _L), _L)
                p = jnp.exp(abuf[sl] - m)
                abuf[sl] = p
                acc = acc + p
            return acc
        s16 = lax.fori_loop(0, na // 4, exp_step,
                            jnp.zeros((_L,), jnp.float32))
        scale = jnp.float32(_K) / _butterfly(s16, jnp.add)

        def aw_step(j, _):
            for u in range(4):
                sl = pl.ds(pl.multiple_of((j * 4 + u) * _L, _L), _L)
                w = jnp.minimum(abuf[sl] * scale, 1.0)
                abuf[sl] = jnp.where(akey[sl] >= Ta, w, 0.0)
            return 0
        lax.fori_loop(0, na // 4, aw_step, 0)
        pltpu.sync_copy(abuf, out_w.at[pl.ds(_NDEL + g * _NCAND, _NCAND)])
        return 0

    lax.fori_loop(0, 4, do_graph, 0)


@jax.jit
def _sc_weights(addl, dell):
    mesh = plsc.VectorSubcoreMesh(core_axis_name="c", subcore_axis_name="s", num_cores=1)
    return pl.kernel(
        _sc_body,
        out_type=jax.ShapeDtypeStruct((_NDEL + _NADD,), jnp.float32),
        mesh=mesh,
        scratch_types=[
            pltpu.VMEM((_NEDGE,), jnp.float32),
            pltpu.VMEM((_NEDGE,), jnp.int32),
            pltpu.VMEM((_NCAND,), jnp.float32),
            pltpu.VMEM((_NCAND,), jnp.int32),
        ],
    )(addl, dell)


def kernel(addition_logits, deletion_logits, edge_candidate_idx, edge_index):
    merged_edge_weight = _sc_weights(
        addition_logits.reshape(_NADD),
        deletion_logits.reshape(_NDEL),
    )
    merged_edge_index = jnp.concatenate(
        [edge_index, edge_candidate_idx.T], axis=1)
    return merged_edge_index, merged_edge_weight


# speedup vs baseline: 1.2350x; 1.2350x over previous
import functools

import jax
import jax.numpy as jnp
from jax import lax
from jax.experimental import pallas as pl
from jax.experimental.pallas import tpu as pltpu
from jax.experimental.pallas import tpu_sc as plsc

_G = 64
_NCAND = 1024
_NEDGE = 2048
_K = 32
_NDEL = _G * _NEDGE
_NADD = _G * _NCAND
_NW = 32
_L = 16
_GSC = 32


_GDN = lax.GatherDimensionNumbers(
    offset_dims=(), collapsed_slice_dims=(0,), start_index_map=(0,))


def _vgather(v, idx):
    return lax.gather(v, idx, _GDN, (1,),
                      mode=lax.GatherScatterMode.PROMISE_IN_BOUNDS)


def _butterfly(v, op):
    lane = lax.iota(jnp.int32, _L)
    for sh in (8, 4, 2, 1):
        v = op(v, _vgather(v, (lane ^ sh)[:, None]))
    return v


def _keys16(x):
    b = lax.bitcast_convert_type(x, jnp.int32)
    return jnp.where(b >= 0, b, b ^ 0x7FFFFFFF)


def _kth_largest(key_ref, nvec, k):
    U = 16

    def count_ge(cand):
        def step(j, acc):
            for u in range(U):
                v = key_ref[pl.ds(pl.multiple_of((j * U + u) * _L, _L), _L)]
                acc = acc + jnp.where(v >= cand, 1, 0)
            return acc
        acc = lax.fori_loop(0, nvec // U, step, jnp.zeros((_L,), jnp.int32))
        return _butterfly(acc, jnp.add)

    zero = jnp.zeros((_L,), jnp.int32)
    T = jnp.where(count_ge(zero) >= k, 0, -2**31).astype(jnp.int32)

    def bit_step(i, T):
        cand = T | (1 << (30 - i))
        return jnp.where(count_ge(cand) >= k, cand, T)
    return lax.fori_loop(0, 31, bit_step, T)


def _sc_body(addl, dell, out_w, dbuf, dkey, abuf, akey):
    nc = 2
    wid = lax.axis_index("s") * nc + lax.axis_index("c")

    def do_graph(t, _):
        g = wid + t

        pltpu.sync_copy(dell.at[pl.ds(g * _NEDGE, _NEDGE)], dbuf)
        nd = _NEDGE // _L

        def dk_step(j, _):
            for u in range(4):
                sl = pl.ds(pl.multiple_of((j * 4 + u) * _L, _L), _L)
                dkey[sl] = ~_keys16(dbuf[sl])
            return 0
        lax.fori_loop(0, nd // 4, dk_step, 0)
        Td = _kth_largest(dkey, nd, _K)

        def dw_step(j, _):
            for u in range(4):
                sl = pl.ds(pl.multiple_of((j * 4 + u) * _L, _L), _L)
                dbuf[sl] = jnp.where(dkey[sl] >= Td, 0.0, 1.0)
            return 0
        lax.fori_loop(0, nd // 4, dw_step, 0)
        pltpu.sync_copy(dbuf, out_w.at[pl.ds(g * _NEDGE, _NEDGE)])

        pltpu.sync_copy(addl.at[pl.ds(g * _NCAND, _NCAND)], abuf)
        na = _NCAND // _L

        def ak_step(j, acc):
            for u in range(4):
                sl = pl.ds(pl.multiple_of((j * 4 + u) * _L, _L), _L)
                x = abuf[sl]
                akey[sl] = _keys16(x)
                acc = jnp.maximum(acc, x)
            return acc
        m16 = lax.fori_loop(0, na // 4, ak_step,
                            jnp.full((_L,), -jnp.inf, jnp.float32))
        m = _butterfly(m16, jnp.maximum)
        Ta = _kth_largest(akey, na, _K)

        def exp_step(j, acc):
            for u in range(4):
                sl = pl.ds(pl.multiple_of((j * 4 + u) * _L, _L), _L)
                p = jnp.exp(abuf[sl] - m)
                abuf[sl] = p
                acc = acc + p
            return acc
        s16 = lax.fori_loop(0, na // 4, exp_step,
                            jnp.zeros((_L,), jnp.float32))
        scale = jnp.float32(_K) / _butterfly(s16, jnp.add)

        def aw_step(j, _):
            for u in range(4):
                sl = pl.ds(pl.multiple_of((j * 4 + u) * _L, _L), _L)
                w = jnp.minimum(abuf[sl] * scale, 1.0)
                abuf[sl] = jnp.where(akey[sl] >= Ta, w, 0.0)
            return 0
        lax.fori_loop(0, na // 4, aw_step, 0)
        pltpu.sync_copy(abuf, out_w.at[pl.ds(_GSC * _NEDGE + g * _NCAND, _NCAND)])
        return 0

    lax.fori_loop(0, 1, do_graph, 0)


@jax.jit
def _sc_weights(addl, dell):
    mesh = plsc.VectorSubcoreMesh(core_axis_name="c", subcore_axis_name="s")
    return pl.kernel(
        _sc_body,
        out_type=jax.ShapeDtypeStruct((_GSC * (_NEDGE + _NCAND),), jnp.float32),
        mesh=mesh,
        scratch_types=[
            pltpu.VMEM((_NEDGE,), jnp.float32),
            pltpu.VMEM((_NEDGE,), jnp.int32),
            pltpu.VMEM((_NCAND,), jnp.float32),
            pltpu.VMEM((_NCAND,), jnp.int32),
        ],
    )(addl, dell)


def _tc_body(add_ref, del_ref, addw_ref, delw_ref):
    ntc = _G - _GSC

    def kth(key, k):
        S = jnp.full((ntc, 1), -2**31, jnp.int32)
        for bit in range(31, -1, -1):
            cand = (S ^ (-2**31)) if bit == 31 else (S | jnp.int32(1 << bit))
            cnt = jnp.sum((key >= cand).astype(jnp.int32), axis=1,
                          keepdims=True)
            S = jnp.where(cnt >= _K, cand, S)
        return S

    def skey(x):
        b = jax.lax.bitcast_convert_type(x, jnp.int32)
        return jnp.where(b >= 0, b, b ^ jnp.int32(0x7FFFFFFF))

    d = del_ref[:]
    dkey = ~skey(d)
    delw_ref[:] = jnp.where(dkey >= kth(dkey, _K), 0.0, 1.0)

    a = add_ref[:]
    akey = skey(a)
    m = jnp.max(a, axis=1, keepdims=True)
    p = jnp.exp(a - m)
    w = jnp.minimum((_K * p) / jnp.sum(p, axis=1, keepdims=True), 1.0)
    addw_ref[:] = jnp.where(akey >= kth(akey, _K), w, 0.0)


def kernel(addition_logits, deletion_logits, edge_candidate_idx, edge_index):
    addl = addition_logits.reshape(_NADD)
    dell = deletion_logits.reshape(_NDEL)
    ntc = _G - _GSC
    sc_w = _sc_weights(addl, dell)
    addw_tc, delw_tc = pl.pallas_call(
        _tc_body,
        out_shape=[
            jax.ShapeDtypeStruct((ntc, _NCAND), jnp.float32),
            jax.ShapeDtypeStruct((ntc, _NEDGE), jnp.float32),
        ],
    )(addl[_GSC * _NCAND:].reshape(ntc, _NCAND),
      dell[_GSC * _NEDGE:].reshape(ntc, _NEDGE))
    merged_edge_weight = jnp.concatenate([
        sc_w[:_GSC * _NEDGE], delw_tc.reshape(-1),
        sc_w[_GSC * _NEDGE:], addw_tc.reshape(-1)])
    merged_edge_index = jnp.concatenate(
        [edge_index, edge_candidate_idx.T], axis=1)
    return merged_edge_index, merged_edge_weight


# speedup vs baseline: 1.3348x; 1.0807x over previous
import jax
import jax.numpy as jnp
from jax import lax
from jax.experimental import pallas as pl
from jax.experimental.pallas import tpu as pltpu
from jax.experimental.pallas import tpu_sc as plsc

_G = 64
_NCAND = 1024
_NEDGE = 2048
_K = 32
_NDEL = _G * _NEDGE
_NADD = _G * _NCAND
_L = 16


_GDN = lax.GatherDimensionNumbers(
    offset_dims=(), collapsed_slice_dims=(0,), start_index_map=(0,))


def _vgather(v, idx):
    return lax.gather(v, idx, _GDN, (1,),
                      mode=lax.GatherScatterMode.PROMISE_IN_BOUNDS)


def _butterfly(v, op):
    lane = lax.iota(jnp.int32, _L)
    for sh in (8, 4, 2, 1):
        v = op(v, _vgather(v, (lane ^ sh)[:, None]))
    return v


def _keys16(x):
    b = lax.bitcast_convert_type(x, jnp.int32)
    return jnp.where(b >= 0, b, b ^ 0x7FFFFFFF)


def _kth_largest(key_ref, nvec, k):
    U = 16

    def count_ge(cand):
        def step(j, acc):
            for u in range(U):
                v = key_ref[pl.ds(pl.multiple_of((j * U + u) * _L, _L), _L)]
                acc = acc + jnp.where(v >= cand, 1, 0)
            return acc
        acc = lax.fori_loop(0, nvec // U, step, jnp.zeros((_L,), jnp.int32))
        return _butterfly(acc, jnp.add)

    zero = jnp.zeros((_L,), jnp.int32)
    T = jnp.where(count_ge(zero) >= k, 0, -2**31).astype(jnp.int32)

    def bit_step(i, T):
        cand = T | (1 << (30 - i))
        return jnp.where(count_ge(cand) >= k, cand, T)
    return lax.fori_loop(0, 31, bit_step, T)


def _sc_body(addl, dell, out_w, dbuf, dkey, abuf, akey):
    nc = 2
    wid = lax.axis_index("s") * nc + lax.axis_index("c")

    def do_graph(t, _):
        g = wid * 2 + t

        pltpu.sync_copy(dell.at[pl.ds(g * _NEDGE, _NEDGE)], dbuf)
        nd = _NEDGE // _L

        def dk_step(j, _):
            for u in range(4):
                sl = pl.ds(pl.multiple_of((j * 4 + u) * _L, _L), _L)
                dkey[sl] = ~_keys16(dbuf[sl])
            return 0
        lax.fori_loop(0, nd // 4, dk_step, 0)
        Td = _kth_largest(dkey, nd, _K)

        def dw_step(j, _):
            for u in range(4):
                sl = pl.ds(pl.multiple_of((j * 4 + u) * _L, _L), _L)
                dbuf[sl] = jnp.where(dkey[sl] >= Td, 0.0, 1.0)
            return 0
        lax.fori_loop(0, nd // 4, dw_step, 0)
        pltpu.sync_copy(dbuf, out_w.at[pl.ds(g * _NEDGE, _NEDGE)])

        pltpu.sync_copy(addl.at[pl.ds(g * _NCAND, _NCAND)], abuf)
        na = _NCAND // _L

        def ak_step(j, acc):
            for u in range(4):
                sl = pl.ds(pl.multiple_of((j * 4 + u) * _L, _L), _L)
                x = abuf[sl]
                akey[sl] = _keys16(x)
                acc = jnp.maximum(acc, x)
            return acc
        m16 = lax.fori_loop(0, na // 4, ak_step,
                            jnp.full((_L,), -jnp.inf, jnp.float32))
        m = _butterfly(m16, jnp.maximum)
        Ta = _kth_largest(akey, na, _K)

        def exp_step(j, acc):
            for u in range(4):
                sl = pl.ds(pl.multiple_of((j * 4 + u) * _L, _L), _L)
                p = jnp.exp(abuf[sl] - m)
                abuf[sl] = p
                acc = acc + p
            return acc
        s16 = lax.fori_loop(0, na // 4, exp_step,
                            jnp.zeros((_L,), jnp.float32))
        scale = jnp.float32(_K) / _butterfly(s16, jnp.add)

        def aw_step(j, _):
            for u in range(4):
                sl = pl.ds(pl.multiple_of((j * 4 + u) * _L, _L), _L)
                w = jnp.minimum(abuf[sl] * scale, 1.0)
                abuf[sl] = jnp.where(akey[sl] >= Ta, w, 0.0)
            return 0
        lax.fori_loop(0, na // 4, aw_step, 0)
        pltpu.sync_copy(abuf, out_w.at[pl.ds(_NDEL + g * _NCAND, _NCAND)])
        return 0

    lax.fori_loop(0, 2, do_graph, 0)


@jax.jit
def _sc_weights(addl, dell):
    mesh = plsc.VectorSubcoreMesh(core_axis_name="c", subcore_axis_name="s")
    return pl.kernel(
        _sc_body,
        out_type=jax.ShapeDtypeStruct((_NDEL + _NADD,), jnp.float32),
        mesh=mesh,
        scratch_types=[
            pltpu.VMEM((_NEDGE,), jnp.float32),
            pltpu.VMEM((_NEDGE,), jnp.int32),
            pltpu.VMEM((_NCAND,), jnp.float32),
            pltpu.VMEM((_NCAND,), jnp.int32),
        ],
    )(addl, dell)


def kernel(addition_logits, deletion_logits, edge_candidate_idx, edge_index):
    merged_edge_weight = _sc_weights(
        addition_logits.reshape(_NADD),
        deletion_logits.reshape(_NDEL),
    )
    merged_edge_index = jnp.concatenate(
        [edge_index, edge_candidate_idx.T], axis=1)
    return merged_edge_index, merged_edge_weight


# speedup vs baseline: 1.3849x; 1.0376x over previous
import jax
import jax.numpy as jnp
from jax import lax
from jax.experimental import pallas as pl
from jax.experimental.pallas import tpu as pltpu
from jax.experimental.pallas import tpu_sc as plsc

_G = 64
_NCAND = 1024
_NEDGE = 2048
_K = 32
_NDEL = _G * _NEDGE
_NADD = _G * _NCAND
_L = 16


_GDN = lax.GatherDimensionNumbers(
    offset_dims=(), collapsed_slice_dims=(0,), start_index_map=(0,))


def _vgather(v, idx):
    return lax.gather(v, idx, _GDN, (1,),
                      mode=lax.GatherScatterMode.PROMISE_IN_BOUNDS)


def _butterfly(v, op):
    lane = lax.iota(jnp.int32, _L)
    for sh in (8, 4, 2, 1):
        v = op(v, _vgather(v, (lane ^ sh)[:, None]))
    return v


def _keys16(x):
    b = lax.bitcast_convert_type(x, jnp.int32)
    return jnp.where(b >= 0, b, b ^ 0x7FFFFFFF)


def _kth_largest(key_ref, nvec, k):
    U = 16

    def count_ge(cand):
        def step(j, acc):
            for u in range(U):
                v = key_ref[pl.ds(pl.multiple_of((j * U + u) * _L, _L), _L)]
                acc = acc + jnp.where(v >= cand, 1, 0)
            return acc
        acc = lax.fori_loop(0, nvec // U, step, jnp.zeros((_L,), jnp.int32))
        return _butterfly(acc, jnp.add)

    zero = jnp.zeros((_L,), jnp.int32)
    T = jnp.where(count_ge(zero) >= k, 0, -2**31).astype(jnp.int32)

    def bit_step(i, T):
        cand = T | (1 << (30 - i))
        return jnp.where(count_ge(cand) >= k, cand, T)
    return lax.fori_loop(0, 31, bit_step, T)


def _sc_body(addl, dell, out_w,
             dbuf0, dbuf1, dkey, abuf0, abuf1, akey,
             sd0, sd1, sa0, sa1, so):
    nc = 2
    wid = lax.axis_index("s") * nc + lax.axis_index("c")

    ins = []
    for t, (db, ab, sd, sa) in enumerate(
            ((dbuf0, abuf0, sd0, sa0), (dbuf1, abuf1, sd1, sa1))):
        g = wid * 2 + t
        ins.append((
            pltpu.async_copy(dell.at[pl.ds(g * _NEDGE, _NEDGE)], db, sd),
            pltpu.async_copy(addl.at[pl.ds(g * _NCAND, _NCAND)], ab, sa)))

    outs = []
    for t, (db, ab, sd, sa) in enumerate(
            ((dbuf0, abuf0, sd0, sa0), (dbuf1, abuf1, sd1, sa1))):
        g = wid * 2 + t

        ins[t][0].wait()
        nd = _NEDGE // _L

        def dk_step(j, _, db=db):
            for u in range(4):
                sl = pl.ds(pl.multiple_of((j * 4 + u) * _L, _L), _L)
                dkey[sl] = ~_keys16(db[sl])
            return 0
        lax.fori_loop(0, nd // 4, dk_step, 0)
        Td = _kth_largest(dkey, nd, _K)

        def dw_step(j, _, db=db, Td=Td):
            for u in range(4):
                sl = pl.ds(pl.multiple_of((j * 4 + u) * _L, _L), _L)
                db[sl] = jnp.where(dkey[sl] >= Td, 0.0, 1.0)
            return 0
        lax.fori_loop(0, nd // 4, dw_step, 0)
        outs.append(pltpu.async_copy(
            db, out_w.at[pl.ds(g * _NEDGE, _NEDGE)], so))

        ins[t][1].wait()
        na = _NCAND // _L

        def ak_step(j, acc, ab=ab):
            for u in range(4):
                sl = pl.ds(pl.multiple_of((j * 4 + u) * _L, _L), _L)
                x = ab[sl]
                akey[sl] = _keys16(x)
                acc = jnp.maximum(acc, x)
            return acc
        m16 = lax.fori_loop(0, na // 4, ak_step,
                            jnp.full((_L,), -jnp.inf, jnp.float32))
        m = _butterfly(m16, jnp.maximum)
        Ta = _kth_largest(akey, na, _K)

        def exp_step(j, acc, ab=ab, m=m):
            for u in range(4):
                sl = pl.ds(pl.multiple_of((j * 4 + u) * _L, _L), _L)
                p = jnp.exp(ab[sl] - m)
                ab[sl] = p
                acc = acc + p
            return acc
        s16 = lax.fori_loop(0, na // 4, exp_step,
                            jnp.zeros((_L,), jnp.float32))
        scale = jnp.float32(_K) / _butterfly(s16, jnp.add)

        def aw_step(j, _, ab=ab, Ta=Ta, scale=scale):
            for u in range(4):
                sl = pl.ds(pl.multiple_of((j * 4 + u) * _L, _L), _L)
                w = jnp.minimum(ab[sl] * scale, 1.0)
                ab[sl] = jnp.where(akey[sl] >= Ta, w, 0.0)
            return 0
        lax.fori_loop(0, na // 4, aw_step, 0)
        outs.append(pltpu.async_copy(
            ab, out_w.at[pl.ds(_NDEL + g * _NCAND, _NCAND)], so))

    for cp in outs:
        cp.wait()


@jax.jit
def _sc_weights(addl, dell):
    mesh = plsc.VectorSubcoreMesh(core_axis_name="c", subcore_axis_name="s")
    return pl.kernel(
        _sc_body,
        out_type=jax.ShapeDtypeStruct((_NDEL + _NADD,), jnp.float32),
        mesh=mesh,
        scratch_types=[
            pltpu.VMEM((_NEDGE,), jnp.float32),
            pltpu.VMEM((_NEDGE,), jnp.float32),
            pltpu.VMEM((_NEDGE,), jnp.int32),
            pltpu.VMEM((_NCAND,), jnp.float32),
            pltpu.VMEM((_NCAND,), jnp.float32),
            pltpu.VMEM((_NCAND,), jnp.int32),
            pltpu.SemaphoreType.DMA,
            pltpu.SemaphoreType.DMA,
            pltpu.SemaphoreType.DMA,
            pltpu.SemaphoreType.DMA,
            pltpu.SemaphoreType.DMA,
        ],
    )(addl, dell)


def kernel(addition_logits, deletion_logits, edge_candidate_idx, edge_index):
    merged_edge_weight = _sc_weights(
        addition_logits.reshape(_NADD),
        deletion_logits.reshape(_NDEL),
    )
    merged_edge_index = jnp.concatenate(
        [edge_index, edge_candidate_idx.T], axis=1)
    return merged_edge_index, merged_edge_weight
